# baseline (speedup 1.0000x reference)
import jax
import jax.numpy as jnp
from jax import lax
from jax.experimental import pallas as pl
from jax.experimental.pallas import tpu as pltpu
from jax.experimental.pallas import tpu_sc as plsc

N = 10000
E = 320000
D = 128
NW = 32
NSEG = 2
NRNG = 16
RPW = 640
NPAD = NRNG * RPW
CH = 1600
NCHW = (E // NSEG) // CH
GB = 64
PEND = CH + 224
EPS = 1e-5


def _stage_a_body(x_ref, w_ref, pb_ref, g_ref, ewmc_ref,
                  ucg_ref, vt_ref, ct_ref, bcg_ref, cv_ref):
    x = x_ref[...]
    w = w_ref[...]
    u = lax.dot_general(x, w, (((1,), (1,)), ((), ())),
                        preferred_element_type=jnp.float32,
                        precision=lax.Precision.HIGHEST)
    mu = jnp.mean(u, axis=1, keepdims=True)
    uc = u - mu
    v = jnp.mean(uc * uc, axis=1, keepdims=True)
    pb = pb_ref[...]
    g = g_ref[...]
    bc = pb - jnp.mean(pb)
    c = jnp.mean(uc * bc, axis=1, keepdims=True)
    ucg_ref[...] = uc * g
    vt_ref[...] = v
    ct_ref[...] = c
    bcg_ref[...] = bc * g
    w0 = ewmc_ref[0, 0]
    ewmc_sp = jnp.log1p(jnp.exp(-jnp.abs(w0))) + jnp.maximum(w0, 0.0)
    vb = jnp.mean(bc * bc)
    cv_ref[...] = jnp.concatenate(
        [jnp.full((1, 16), ewmc_sp, jnp.float32),
         jnp.full((1, 16), vb, jnp.float32)], axis=0)


def _stage_a(xp, pool_w, pool_b, lnp_g, ewmc):
    return pl.pallas_call(
        _stage_a_body,
        out_shape=(
            jax.ShapeDtypeStruct((NPAD, D), jnp.float32),
            jax.ShapeDtypeStruct((NPAD, 1), jnp.float32),
            jax.ShapeDtypeStruct((NPAD, 1), jnp.float32),
            jax.ShapeDtypeStruct((1, D), jnp.float32),
            jax.ShapeDtypeStruct((2, 16), jnp.float32),
        ),
    )(xp, pool_w, pool_b.reshape(1, D), lnp_g.reshape(1, D),
      ewmc.reshape(1, 1))


_GDN = lax.GatherDimensionNumbers(offset_dims=(), collapsed_slice_dims=(0,),
                                  start_index_map=(0,))


def _permute(x, idx):
    return lax.gather(x, idx[:, None], _GDN, slice_sizes=(1,),
                      mode=lax.GatherScatterMode.PROMISE_IN_BOUNDS)

def _rsqrt_bits(z):
    i = plsc.bitcast(z, jnp.int32)
    i = jnp.int32(0x5F3759DF) - (i >> 1)
    y = plsc.bitcast(i, jnp.float32)
    for _ in range(3):
        y = y * (1.5 - 0.5 * z * y * y)
    return y


def _stage_b_body(ucg_hbm, vt_hbm, ct_hbm, bcg_hbm, lnb_hbm, cv_hbm,
                  src_hbm, dst_hbm, ew_hbm, out_hbm,
                  acc, dstb, srcb, ewb, dstb2, srcb2, ewb2, bms,
                  p_src, p_ldst, p_ew, p_p, p_q,
                  rows, rows2, vch, vch2, cch, cch2,
                  bcg_v, lnb_v, cv_v, tbl, sem, sem2):
    wid = lax.axis_index("s") * 2 + lax.axis_index("c")
    seg = wid & 1
    rng = wid >> 1
    lo = rng * RPW
    ebase = seg * (E // NSEG)

    pltpu.sync_copy(bcg_hbm, bcg_v)
    pltpu.sync_copy(lnb_hbm, lnb_v)
    pltpu.sync_copy(cv_hbm, cv_v)

    for i, tv in enumerate([0, 1, 28, 2, 29, 14, 24, 3, 30, 22, 20, 15, 25,
                            17, 4, 8, 31, 27, 13, 23, 21, 19, 16, 7, 26, 12,
                            18, 6, 11, 5, 10, 9]):
        tbl[i] = tv

    zero16f = jnp.zeros((16,), jnp.float32)

    @plsc.parallel_loop(0, RPW + 1, unroll=4)
    def _zrow(r):
        for j in range(8):
            acc[r, pl.ds(16 * j, 16)] = zero16f

    ewmc = cv_v[pl.ds(0, 16)]
    vb = cv_v[pl.ds(16, 16)]
    lane = lax.iota(jnp.int32, 16)
    zeros16 = jnp.zeros((16,), jnp.int32)
    ones16 = jnp.ones((16,), jnp.int32)
    f15 = jnp.full((16,), 15, jnp.int32)
    c26 = jnp.full((16,), 26, jnp.int32)
    lov = jnp.full((16,), lo, jnp.int32)
    rv = jnp.full((16,), RPW, jnp.int32)
    tree_idx = [jnp.minimum(lane + jnp.full((16,), st, jnp.int32), f15)
                for st in (1, 2, 4, 8)]
    cEPS = jnp.full((16,), EPS, jnp.float32)
    c1 = jnp.full((16,), 1.0, jnp.float32)
    c2 = jnp.full((16,), 2.0, jnp.float32)
    cH = jnp.full((16,), 0.5, jnp.float32)

    def _cf(x):
        return jnp.full((16,), x, jnp.float32)

    rungs = [(_cf(10.0 ** p), _cf(10.0 ** ((p - 1) / 2.0 + 0.75)))
             for p in (-4, -2, 0, 2, 4)]

    def _rsqrt16(z):
        y = _cf(10.0 ** 2.75)
        for thr, seed in reversed(rungs):
            y = jnp.where(z < thr, seed, y)
        for _ in range(6):
            y = cH * (y + z / y)
        return c1 / y

    bcg8 = [bcg_v[pl.ds(16 * j, 16)] for j in range(8)]
    lnb8 = [lnb_v[pl.ds(16 * j, 16)] for j in range(8)]

    def _load3(ci, db, sb, eb):
        cc = ebase + ci * CH
        a1 = pltpu.async_copy(dst_hbm.at[pl.ds(cc, CH)], db.at[pl.ds(0, CH)], sem2)
        a2 = pltpu.async_copy(src_hbm.at[pl.ds(cc, CH)], sb.at[pl.ds(0, CH)], sem2)
        a3 = pltpu.async_copy(ew_hbm.at[pl.ds(cc, CH)], eb.at[pl.ds(0, CH)], sem2)
        a1.wait(); a2.wait(); a3.wait()

    def _process(dstc, srcc, ewc):
        @plsc.parallel_loop(0, CH // 16, unroll=4)
        def _p1(v):
            d = dstc[pl.ds(16 * v, 16)]
            ld = d - lov
            m = (ld >= zeros16) & (ld < rv)
            mi = jnp.where(m, ones16, zeros16)
            bits = (mi << lane) + (mi << c26)
            for k in range(4):
                bits = bits + _permute(bits, tree_idx[k])
            bms[pl.ds(16 * v, 16)] = bits

        def _match_vreg(v, r0, cnt):
            bm0 = r0 & 0xFFFF
            kcnt = lax.shift_right_logical(r0, 26)
            d = dstc[pl.ds(16 * v, 16)]
            sv = srcc[pl.ds(16 * v, 16)]
            ev = ewc[pl.ds(16 * v, 16)]
            ld = d - lov

            def _mb(t, carry):
                bm, cn = carry
                lsb = bm & (0 - bm)
                dbi = lax.shift_right_logical(lsb * 0x077CB531, 27) & 31
                jv = jnp.full((16,), tbl[dbi], jnp.int32)
                p_src[pl.ds(cn, 16)] = _permute(sv, jv)
                p_ldst[pl.ds(cn, 16)] = _permute(ld, jv)
                p_ew[pl.ds(cn, 16)] = _permute(ev, jv)
                return bm & (bm - 1), cn + 1

            _, cnt = lax.fori_loop(0, kcnt, _mb, (bm0, cnt))
            return cnt

        def _p2(t, cnt):
            r0a = bms[pl.ds(64 * t, 16)][0]
            r0b = bms[pl.ds(64 * t + 16, 16)][0]
            r0c = bms[pl.ds(64 * t + 32, 16)][0]
            r0d = bms[pl.ds(64 * t + 48, 16)][0]
            cnt = _match_vreg(4 * t, r0a, cnt)
            cnt = _match_vreg(4 * t + 1, r0b, cnt)
            cnt = _match_vreg(4 * t + 2, r0c, cnt)
            cnt = _match_vreg(4 * t + 3, r0d, cnt)
            return cnt

        mcount = lax.fori_loop(0, CH // 64, _p2, 0)

        for k in range(12):
            p_src[pl.ds(mcount + 16 * k, 16)] = zeros16

        nb = (mcount + GB - 1) // GB

        def _gload(b, rbuf, vbuf, cbuf):
            idx = p_src.at[pl.ds(b * GB, GB)]
            a1 = pltpu.async_copy(ucg_hbm.at[idx], rbuf, sem)
            a2 = pltpu.async_copy(vt_hbm.at[idx], vbuf, sem)
            a3 = pltpu.async_copy(ct_hbm.at[idx], cbuf, sem)
            a1.wait(); a2.wait(); a3.wait()

        def _consume(b, rbuf, vbuf, cbuf):
            for t in range(GB // 16):
                ev = p_ew[pl.ds(b * GB + 16 * t, 16)]
                vv = vbuf[pl.ds(16 * t, 16)]
                cc = cbuf[pl.ds(16 * t, 16)]
                sc = c1 + ewmc * ev
                varw = sc * sc * vv + c2 * sc * cc + vb + cEPS
                inv = _rsqrt16(varw)
                p_p[pl.ds(b * GB + 16 * t, 16)] = sc * inv
                p_q[pl.ds(b * GB + 16 * t, 16)] = inv

            mcv = jnp.full((16,), mcount, jnp.int32)
            trashv = jnp.full((16,), RPW, jnp.int32)

            def _e16(t, _):
                off = b * GB + 16 * t
                ldr16 = p_ldst[pl.ds(off, 16)]
                ev16 = jnp.full((16,), off, jnp.int32) + lane
                ldr16 = jnp.where(ev16 < mcv, ldr16, trashv)
                pp16 = p_p[pl.ds(off, 16)]
                qq16 = p_q[pl.ds(off, 16)]
                for i in range(16):
                    li = jnp.full((16,), i, jnp.int32)
                    ppv = _permute(pp16, li)
                    qqv = _permute(qq16, li)
                    ldr = ldr16[i]
                    ri = 16 * t + i
                    row8 = [rbuf[ri, pl.ds(16 * j, 16)] for j in range(8)]
                    val8 = [ppv * row8[j] + (qqv * bcg8[j] + lnb8[j])
                            for j in range(8)]
                    a8 = [acc[ldr, pl.ds(16 * j, 16)] for j in range(8)]
                    for j in range(8):
                        acc[ldr, pl.ds(16 * j, 16)] = jnp.maximum(a8[j], val8[j])
                return 0
            nt = (jnp.minimum(mcount - b * GB, GB) + 15) // 16
            lax.fori_loop(0, nt, _e16, 0)

        def _batch(b, _):
            _gload(b, rows, vch, cch)
            _consume(b, rows, vch, cch)
            return 0
        lax.fori_loop(0, nb, _batch, 0)

    def _chunk(ci, _):
        _load3(ci, dstb, srcb, ewb)
        _process(dstb, srcb, ewb)
        return 0

    lax.fori_loop(0, NCHW, _chunk, 0)
    pltpu.sync_copy(acc.at[pl.ds(0, RPW)],
                    out_hbm.at[pl.ds(seg * NPAD + lo, RPW)])


def _stage_b(ucg, vt, ct, bcg, lnb, cv, src, dst, ew):
    mesh = plsc.VectorSubcoreMesh(core_axis_name="c", subcore_axis_name="s")
    f = pl.kernel(
        _stage_b_body,
        out_type=jax.ShapeDtypeStruct((NSEG * NPAD, D), jnp.float32),
        mesh=mesh,
        scratch_types=[
            pltpu.VMEM((RPW + 1, D), jnp.float32),
            pltpu.VMEM((CH + 16,), jnp.int32),
            pltpu.VMEM((CH + 16,), jnp.int32),
            pltpu.VMEM((CH + 16,), jnp.float32),
            pltpu.VMEM((CH + 16,), jnp.int32),
            pltpu.VMEM((CH + 16,), jnp.int32),
            pltpu.VMEM((CH + 16,), jnp.float32),
            pltpu.VMEM((CH,), jnp.int32),
            pltpu.VMEM((PEND,), jnp.int32),
            pltpu.VMEM((PEND,), jnp.int32),
            pltpu.VMEM((PEND,), jnp.float32),
            pltpu.VMEM((PEND,), jnp.float32),
            pltpu.VMEM((PEND,), jnp.float32),
            pltpu.VMEM((GB, D), jnp.float32),
            pltpu.VMEM((GB, D), jnp.float32),
            pltpu.VMEM((GB,), jnp.float32),
            pltpu.VMEM((GB,), jnp.float32),
            pltpu.VMEM((GB,), jnp.float32),
            pltpu.VMEM((GB,), jnp.float32),
            pltpu.VMEM((D,), jnp.float32),
            pltpu.VMEM((D,), jnp.float32),
            pltpu.VMEM((32,), jnp.float32),
            pltpu.SMEM((32,), jnp.int32),
            pltpu.SemaphoreType.DMA,
            pltpu.SemaphoreType.DMA,
        ],
    )
    return f(ucg, vt, ct, bcg, lnb, cv, src, dst, ew)


def _ln(v, g, b):
    m = jnp.mean(v, axis=-1, keepdims=True)
    var = jnp.mean((v - m) ** 2, axis=-1, keepdims=True)
    return (v - m) * lax.rsqrt(var + EPS) * g + b


def _stage_c_body(agg_ref, g0_ref, b0_ref, w1_ref, b1_ref, g1_ref, bb1_ref,
                  mw_ref, mb_ref, lw_ref, lb_ref, mu_ref, std_ref):
    ag = agg_ref[...]
    a = jnp.maximum(ag[:NPAD], ag[NPAD:])
    h = _ln(a, g0_ref[...], b0_ref[...])
    h = lax.dot_general(h, w1_ref[...], (((1,), (1,)), ((), ())),
                        preferred_element_type=jnp.float32,
                        precision=lax.Precision.HIGHEST) + b1_ref[...]
    h = _ln(h, g1_ref[...], bb1_ref[...])
    h = jnp.maximum(h, 0.0)
    mu_ref[...] = lax.dot_general(h, mw_ref[...], (((1,), (1,)), ((), ())),
                                  preferred_element_type=jnp.float32,
                                  precision=lax.Precision.HIGHEST) + mb_ref[...]
    lv = lax.dot_general(h, lw_ref[...], (((1,), (1,)), ((), ())),
                         preferred_element_type=jnp.float32,
                         precision=lax.Precision.HIGHEST) + lb_ref[...]
    std_ref[...] = jnp.exp(0.5 * lv)


def _stage_c(agg, ln0_g, ln0_b, w1, b1, ln1_g, ln1_b, mu_w, mu_b, lv_w, lv_b):
    return pl.pallas_call(
        _stage_c_body,
        out_shape=(
            jax.ShapeDtypeStruct((NPAD, D), jnp.float32),
            jax.ShapeDtypeStruct((NPAD, D), jnp.float32),
        ),
    )(agg, ln0_g.reshape(1, D), ln0_b.reshape(1, D), w1,
      b1.reshape(1, D), ln1_g.reshape(1, D), ln1_b.reshape(1, D),
      mu_w, mu_b.reshape(1, D), lv_w, lv_b.reshape(1, D))


@jax.jit
def kernel(x, edge_index, edge_weight, ewmc, pool_W, pool_b, lnp_g, lnp_b,
           ln0_g, ln0_b, W1, b1, ln1_g, ln1_b, mu_W, mu_b, lv_W, lv_b):
    xp = jnp.pad(x.astype(jnp.float32), ((0, NPAD - N), (0, 0)))
    src = edge_index[0].astype(jnp.int32)
    dst = edge_index[1].astype(jnp.int32)
    ew = edge_weight.astype(jnp.float32)

    ucg, vt, ct, bcg, cv = _stage_a(
        xp, pool_W, pool_b, lnp_g, jnp.asarray(ewmc, jnp.float32))
    agg = _stage_b(ucg, vt.reshape(NPAD), ct.reshape(NPAD), bcg.reshape(D),
                   lnp_b.reshape(D).astype(jnp.float32), cv.reshape(32),
                   src, dst, ew)
    mu, std = _stage_c(agg, ln0_g, ln0_b, W1, b1, ln1_g, ln1_b,
                       mu_W, mu_b, lv_W, lv_b)
    return mu[:N], std[:N]
